# baseline (speedup 1.0000x reference)
import functools

import jax
import jax.numpy as jnp
from jax import lax
from jax.experimental import pallas as pl
from jax.experimental.pallas import tpu as pltpu
from jax.experimental.pallas import tpu_sc as plsc

NC = 2
NS = 16
L = 16
CH = 64
WR = 256


def _matmul_body(x_ref, w_ref, o_ref):
    o_ref[...] = jnp.dot(x_ref[...], w_ref[...],
                         preferred_element_type=jnp.float32)


def _tc_matmul(xf, W):
    BN, D_in = xf.shape
    D_out = W.shape[1]
    BLK = 1024
    return pl.pallas_call(
        _matmul_body,
        grid=(BN // BLK,),
        in_specs=[
            pl.BlockSpec((BLK, D_in), lambda i: (i, 0)),
            pl.BlockSpec((D_in, D_out), lambda i: (0, 0)),
        ],
        out_specs=pl.BlockSpec((BLK, D_out), lambda i: (i, 0)),
        out_shape=jax.ShapeDtypeStruct((BN, D_out), jnp.float32),
    )(xf, W)


def _gcn_sc(xw, rows, cols, wts, bias, B, N, D):
    BN = B * N
    E_pad = rows.shape[0]
    EPS = E_pad // NS
    G = EPS // CH
    RPT = N // NS
    BPC = B // NC
    NH = RPT // CH
    FV = D // L
    IT = max(EPS - 1, 1).bit_length() + 1

    mesh = plsc.VectorSubcoreMesh(core_axis_name="c", subcore_axis_name="s",
                                  num_cores=NC, num_subcores=NS)

    scratch = [
        pltpu.VMEM((EPS,), jnp.int32),
        pltpu.VMEM((EPS,), jnp.int32),
        pltpu.VMEM((EPS,), jnp.float32),
        pltpu.VMEM((WR, D), jnp.float32),
        pltpu.VMEM((CH, D), jnp.float32),
        pltpu.VMEM((CH,), jnp.int32),
        pltpu.VMEM((CH,), jnp.float32),
        pltpu.VMEM((CH,), jnp.int32),
        pltpu.VMEM((N,), jnp.float32),
        pltpu.VMEM((RPT,), jnp.float32),
        pltpu.VMEM((D,), jnp.float32),
        pltpu.VMEM_SHARED((N,), jnp.float32),
        pltpu.VMEM_SHARED((N,), jnp.float32),
        pltpu.VMEM_SHARED((N, D), jnp.float32),
    ]

    @functools.partial(
        pl.kernel,
        out_type=jax.ShapeDtypeStruct((BN, D), jnp.float32),
        mesh=mesh,
        compiler_params=pltpu.CompilerParams(needs_layout_passes=False),
        scratch_types=scratch,
    )
    def k(xw_hbm, rows_hbm, cols_hbm, w_hbm, b_hbm, out_hbm,
          er, ec, ew, xwin, mb, sidb, wch, rch, dinv_loc, degs, bloc,
          deg_sh, dinv_sh, acc):
        sid = lax.axis_index("s")
        cid = lax.axis_index("c")
        ebase = sid * EPS
        zeros = jnp.zeros((L,), jnp.float32)

        pltpu.sync_copy(rows_hbm.at[pl.ds(ebase, EPS)], er)
        pltpu.sync_copy(cols_hbm.at[pl.ds(ebase, EPS)], ec)
        pltpu.sync_copy(w_hbm.at[pl.ds(ebase, EPS)], ew)
        pltpu.sync_copy(b_hbm, bloc)

        def zdeg(i, _):
            degs[pl.ds(i * L, L)] = zeros
            return 0
        lax.fori_loop(0, RPT // L, zdeg, 0)
        pltpu.sync_copy(degs, deg_sh.at[pl.ds(sid * RPT, RPT)])
        plsc.subcore_barrier()

        def deg_chunk(g, _):
            for j in range(CH // L):
                sidb[pl.ds(j * L, L)] = ec[pl.ds(g * CH + j * L, L)]
                wch[pl.ds(j * L, L)] = ew[pl.ds(g * CH + j * L, L)]
            pltpu.sync_copy(wch, deg_sh.at[sidb], add=True)
            return 0
        lax.fori_loop(0, G, deg_chunk, 0)
        plsc.subcore_barrier()

        pltpu.sync_copy(deg_sh.at[pl.ds(sid * RPT, RPT)], degs)

        def rsqrt_vec(i, _):
            d = degs[pl.ds(i * L, L)] + 1.0
            ib = lax.bitcast_convert_type(d, jnp.int32)
            y = lax.bitcast_convert_type(
                jnp.full((L,), 0x5F3759DF, jnp.int32) - (ib >> 1), jnp.float32)
            for _ in range(3):
                y = y * (1.5 - 0.5 * d * y * y)
            degs[pl.ds(i * L, L)] = y
            return 0
        lax.fori_loop(0, RPT // L, rsqrt_vec, 0)
        pltpu.sync_copy(degs, dinv_sh.at[pl.ds(sid * RPT, RPT)])
        plsc.subcore_barrier()
        pltpu.sync_copy(dinv_sh, dinv_loc)

        v0 = er[pl.ds(0, L)]
        r_lo = (v0[0] // 8) * 8
        v1 = er[pl.ds(EPS - L, L)]
        r_hi = v1[L - 1]
        nwin = (r_hi - r_lo) // WR + 1

        def lower_bound(t):
            lo = jnp.int32(0)
            hi = jnp.int32(EPS)
            for _ in range(IT):
                mid = (lo + hi) // 2
                midc = jnp.minimum(mid, EPS - 1)
                v = plsc.load_gather(
                    er, [jnp.broadcast_to(midc, (L,)).astype(jnp.int32)])[0]
                go = jnp.logical_and(lo < hi, v < t)
                sh = jnp.logical_and(lo < hi, jnp.logical_not(v < t))
                lo = jnp.where(go, mid + 1, lo)
                hi = jnp.where(sh, mid, hi)
            return lo

        iotav = lax.iota(jnp.int32, L)
        bbase = cid * BPC * N

        for lb in range(BPC):
            boff = bbase + lb * N

            for h in range(NH):
                start = sid * RPT + h * CH
                pltpu.sync_copy(xw_hbm.at[pl.ds(boff + start, CH)], mb)

                def ig(j, _):
                    dv = dinv_loc[pl.ds(start + j * L, L)]
                    for u in range(L):
                        dd = dv[u] * dv[u]
                        e = j * L + u
                        for f in range(FV):
                            mb[e, pl.ds(f * L, L)] = (
                                mb[e, pl.ds(f * L, L)] * dd)
                    return 0
                lax.fori_loop(0, CH // L, ig, 0)
                pltpu.sync_copy(mb, acc.at[pl.ds(start, CH)])
            plsc.subcore_barrier()

            def win_body(w, e_lo):
                wbase = r_lo + w * WR
                wstart = pl.multiple_of(boff + wbase, 8)
                pltpu.sync_copy(xw_hbm.at[pl.ds(wstart, WR)], xwin)
                e_hi = lower_bound(wbase + WR)
                e0 = (e_lo // L) * L
                ncb = (e_hi - e0 + CH - 1) // CH

                def ch_body(ck, _):
                    cb = e0 + ck * CH

                    def grp(j, _):
                        off = cb + j * L
                        offc = jnp.minimum(off, EPS - L)
                        r16 = er[pl.ds(offc, L)]
                        c16 = ec[pl.ds(offc, L)]
                        w16 = ew[pl.ds(offc, L)]
                        pos = off + iotav
                        msk = jnp.logical_and(pos >= e_lo, pos < e_hi)
                        dr = plsc.load_gather(dinv_loc, [r16])
                        dc = plsc.load_gather(dinv_loc, [c16])
                        wn = jnp.where(msk, w16 * dr * dc, 0.0)
                        rl = jnp.minimum(jnp.maximum(r16 - wbase, 0), WR - 1)
                        sidb[pl.ds(j * L, L)] = c16
                        for u in range(L):
                            wsc = wn[u]
                            rsc = rl[u]
                            e = j * L + u
                            for f in range(FV):
                                mb[e, pl.ds(f * L, L)] = (
                                    xwin[rsc, pl.ds(f * L, L)] * wsc)
                        return 0
                    lax.fori_loop(0, CH // L, grp, 0)
                    pltpu.sync_copy(mb, acc.at[sidb], add=True)
                    return 0
                lax.fori_loop(0, ncb, ch_body, 0)
                return e_hi
            lax.fori_loop(0, nwin, win_body, jnp.int32(0))
            plsc.subcore_barrier()

            batch = cid * BPC + lb
            for h in range(NH):
                start = sid * RPT + h * CH
                pltpu.sync_copy(acc.at[pl.ds(start, CH)], mb)

                def wout(e, _):
                    for f in range(FV):
                        mb[e, pl.ds(f * L, L)] = (
                            mb[e, pl.ds(f * L, L)] + bloc[pl.ds(f * L, L)])
                    return 0
                lax.fori_loop(0, CH, wout, 0)
                pltpu.sync_copy(mb, out_hbm.at[pl.ds(batch * N + start, CH)])

    return k(xw, rows, cols, wts, bias)


def kernel(x, edge_index, edge_attr, W, b):
    B, N, _ = x.shape
    D_out = W.shape[1]
    E = edge_attr.shape[0]

    xf = x.reshape(B * N, -1)
    xw = _tc_matmul(xf, W)
    xw = jnp.concatenate(
        [xw, jnp.zeros((WR, xw.shape[1]), jnp.float32)], axis=0)

    rows = edge_index[0].astype(jnp.int32)
    cols = edge_index[1].astype(jnp.int32)
    wts = edge_attr.astype(jnp.float32)
    quant = NS * CH
    e_pad = max(((E + quant - 1) // quant) * quant, quant)
    pad = e_pad - E
    rows = jnp.pad(rows, (0, pad), constant_values=N - 1)
    cols = jnp.pad(cols, (0, pad))
    wts = jnp.pad(wts, (0, pad))

    out = _gcn_sc(xw, rows, cols, wts, b, B, N, D_out)
    return out.reshape(B, N, D_out)


# speedup vs baseline: 1.0552x; 1.0552x over previous
import functools

import jax
import jax.numpy as jnp
from jax import lax
from jax.experimental import pallas as pl
from jax.experimental.pallas import tpu as pltpu
from jax.experimental.pallas import tpu_sc as plsc

NC = 2
NS = 16
L = 16
CH = 64
WR = 256


def _matmul_body(x_ref, w_ref, o_ref):
    o_ref[...] = jnp.dot(x_ref[...], w_ref[...],
                         preferred_element_type=jnp.float32)


def _tc_matmul(xf, W):
    BN, D_in = xf.shape
    D_out = W.shape[1]
    BLK = 1024
    return pl.pallas_call(
        _matmul_body,
        grid=(BN // BLK,),
        in_specs=[
            pl.BlockSpec((BLK, D_in), lambda i: (i, 0)),
            pl.BlockSpec((D_in, D_out), lambda i: (0, 0)),
        ],
        out_specs=pl.BlockSpec((BLK, D_out), lambda i: (i, 0)),
        out_shape=jax.ShapeDtypeStruct((BN, D_out), jnp.float32),
    )(xf, W)


def _gcn_sc(xw, rows, cols, wts, bias, B, N, D):
    BN = B * N
    E_pad = rows.shape[0]
    EPS = E_pad // NS
    G = EPS // CH
    RPT = N // NS
    BPC = B // NC
    NH = RPT // CH
    FV = D // L
    IT = max(EPS - 1, 1).bit_length() + 1

    mesh = plsc.VectorSubcoreMesh(core_axis_name="c", subcore_axis_name="s",
                                  num_cores=NC, num_subcores=NS)

    scratch = [
        pltpu.VMEM((EPS,), jnp.int32),
        pltpu.VMEM((EPS,), jnp.int32),
        pltpu.VMEM((EPS,), jnp.float32),
        pltpu.VMEM((WR, D), jnp.float32),
        pltpu.VMEM((CH, D), jnp.float32),
        pltpu.VMEM((CH, D), jnp.float32),
        pltpu.VMEM((CH,), jnp.int32),
        pltpu.VMEM((CH,), jnp.int32),
        pltpu.VMEM((CH,), jnp.float32),
        pltpu.SemaphoreType.DMA,
        pltpu.SemaphoreType.DMA,
        pltpu.VMEM((N,), jnp.float32),
        pltpu.VMEM((RPT,), jnp.float32),
        pltpu.VMEM((D,), jnp.float32),
        pltpu.VMEM_SHARED((N,), jnp.float32),
        pltpu.VMEM_SHARED((N,), jnp.float32),
        pltpu.VMEM_SHARED((N, D), jnp.float32),
    ]

    @functools.partial(
        pl.kernel,
        out_type=jax.ShapeDtypeStruct((BN, D), jnp.float32),
        mesh=mesh,
        compiler_params=pltpu.CompilerParams(needs_layout_passes=False),
        scratch_types=scratch,
    )
    def k(xw_hbm, rows_hbm, cols_hbm, w_hbm, b_hbm, out_hbm,
          er, ec, ew, xwin, mb, mb1, sidb, sidb1, wch, ss0, ss1,
          dinv_loc, degs, bloc, deg_sh, dinv_sh, acc):
        sid = lax.axis_index("s")
        cid = lax.axis_index("c")
        ebase = sid * EPS
        zeros = jnp.zeros((L,), jnp.float32)

        pltpu.sync_copy(rows_hbm.at[pl.ds(ebase, EPS)], er)
        pltpu.sync_copy(cols_hbm.at[pl.ds(ebase, EPS)], ec)
        pltpu.sync_copy(w_hbm.at[pl.ds(ebase, EPS)], ew)
        pltpu.sync_copy(b_hbm, bloc)

        def zdeg(i, _):
            degs[pl.ds(i * L, L)] = zeros
            return 0
        lax.fori_loop(0, RPT // L, zdeg, 0)
        pltpu.sync_copy(degs, deg_sh.at[pl.ds(sid * RPT, RPT)])
        plsc.subcore_barrier()

        def deg_chunk(g, _):
            for j in range(CH // L):
                sidb[pl.ds(j * L, L)] = ec[pl.ds(g * CH + j * L, L)]
                wch[pl.ds(j * L, L)] = ew[pl.ds(g * CH + j * L, L)]
            pltpu.sync_copy(wch, deg_sh.at[sidb], add=True)
            return 0
        lax.fori_loop(0, G, deg_chunk, 0)
        plsc.subcore_barrier()

        pltpu.sync_copy(deg_sh.at[pl.ds(sid * RPT, RPT)], degs)

        def rsqrt_vec(i, _):
            d = degs[pl.ds(i * L, L)] + 1.0
            ib = lax.bitcast_convert_type(d, jnp.int32)
            y = lax.bitcast_convert_type(
                jnp.full((L,), 0x5F3759DF, jnp.int32) - (ib >> 1), jnp.float32)
            for _ in range(3):
                y = y * (1.5 - 0.5 * d * y * y)
            degs[pl.ds(i * L, L)] = y
            return 0
        lax.fori_loop(0, RPT // L, rsqrt_vec, 0)
        pltpu.sync_copy(degs, dinv_sh.at[pl.ds(sid * RPT, RPT)])
        plsc.subcore_barrier()
        pltpu.sync_copy(dinv_sh, dinv_loc)

        v0 = er[pl.ds(0, L)]
        r_lo = (v0[0] // 8) * 8
        v1 = er[pl.ds(EPS - L, L)]
        r_hi = v1[L - 1]
        nwin = (r_hi - r_lo) // WR + 1

        def lower_bound(t):
            lo = jnp.int32(0)
            hi = jnp.int32(EPS)
            for _ in range(IT):
                mid = (lo + hi) // 2
                midc = jnp.minimum(mid, EPS - 1)
                v = plsc.load_gather(
                    er, [jnp.broadcast_to(midc, (L,)).astype(jnp.int32)])[0]
                go = jnp.logical_and(lo < hi, v < t)
                sh = jnp.logical_and(lo < hi, jnp.logical_not(v < t))
                lo = jnp.where(go, mid + 1, lo)
                hi = jnp.where(sh, mid, hi)
            return lo

        iotav = lax.iota(jnp.int32, L)
        bbase = cid * BPC * N

        for lb in range(BPC):
            boff = bbase + lb * N

            for h in range(NH):
                start = sid * RPT + h * CH
                pltpu.sync_copy(xw_hbm.at[pl.ds(boff + start, CH)], mb)

                def ig(j, _):
                    dv = dinv_loc[pl.ds(start + j * L, L)]
                    for u in range(L):
                        dd = dv[u] * dv[u]
                        e = j * L + u
                        for f in range(FV):
                            mb[e, pl.ds(f * L, L)] = (
                                mb[e, pl.ds(f * L, L)] * dd)
                    return 0
                lax.fori_loop(0, CH // L, ig, 0)
                pltpu.sync_copy(mb, acc.at[pl.ds(start, CH)])
            plsc.subcore_barrier()

            def win_body(w, e_lo):
                wbase = r_lo + w * WR
                wstart = pl.multiple_of(boff + wbase, 8)
                pltpu.sync_copy(xw_hbm.at[pl.ds(wstart, WR)], xwin)
                e_hi = lower_bound(wbase + WR)
                e0 = (e_lo // L) * L
                ncb = (e_hi - e0 + CH - 1) // CH

                def grp_into(mbx, sidx, cb):
                    def grp(j, _):
                        off = cb + j * L
                        offc = jnp.minimum(off, EPS - L)
                        r16 = er[pl.ds(offc, L)]
                        c16 = ec[pl.ds(offc, L)]
                        w16 = ew[pl.ds(offc, L)]
                        pos = off + iotav
                        msk = jnp.logical_and(pos >= e_lo, pos < e_hi)
                        dr = plsc.load_gather(dinv_loc, [r16])
                        dc = plsc.load_gather(dinv_loc, [c16])
                        wn = jnp.where(msk, w16 * dr * dc, 0.0)
                        rl = jnp.minimum(jnp.maximum(r16 - wbase, 0), WR - 1)
                        sidx[pl.ds(j * L, L)] = c16
                        for u in range(L):
                            wsc = wn[u]
                            rsc = rl[u]
                            e = j * L + u
                            for f in range(FV):
                                mbx[e, pl.ds(f * L, L)] = (
                                    xwin[rsc, pl.ds(f * L, L)] * wsc)
                        return 0
                    lax.fori_loop(0, CH // L, grp, 0)

                ncb2 = (ncb + 1) // 2

                def ch2_body(ck2, _):
                    @pl.when(ck2 >= 1)
                    def _():
                        pltpu.make_async_copy(mb, acc.at[sidb], ss0).wait()
                        pltpu.make_async_copy(mb1, acc.at[sidb1], ss1).wait()
                    grp_into(mb, sidb, e0 + (ck2 * 2) * CH)
                    pltpu.async_copy(mb, acc.at[sidb], ss0, add=True)

                    @pl.when(ck2 * 2 + 1 < ncb)
                    def _():
                        grp_into(mb1, sidb1, e0 + (ck2 * 2 + 1) * CH)
                        pltpu.async_copy(mb1, acc.at[sidb1], ss1, add=True)
                    return 0
                lax.fori_loop(0, ncb2, ch2_body, 0)

                @pl.when(ncb >= 1)
                def _():
                    pltpu.make_async_copy(mb, acc.at[sidb], ss0).wait()

                @pl.when(jnp.logical_and(ncb >= 2, (ncb % 2) == 0))
                def _():
                    pltpu.make_async_copy(mb1, acc.at[sidb1], ss1).wait()
                return e_hi
            lax.fori_loop(0, nwin, win_body, jnp.int32(0))
            plsc.subcore_barrier()

            batch = cid * BPC + lb
            for h in range(NH):
                start = sid * RPT + h * CH
                pltpu.sync_copy(acc.at[pl.ds(start, CH)], mb)

                def wout(e, _):
                    for f in range(FV):
                        mb[e, pl.ds(f * L, L)] = (
                            mb[e, pl.ds(f * L, L)] + bloc[pl.ds(f * L, L)])
                    return 0
                lax.fori_loop(0, CH, wout, 0)
                pltpu.sync_copy(mb, out_hbm.at[pl.ds(batch * N + start, CH)])

    return k(xw, rows, cols, wts, bias)


def kernel(x, edge_index, edge_attr, W, b):
    B, N, _ = x.shape
    D_out = W.shape[1]
    E = edge_attr.shape[0]

    xf = x.reshape(B * N, -1)
    xw = _tc_matmul(xf, W)
    xw = jnp.concatenate(
        [xw, jnp.zeros((WR, xw.shape[1]), jnp.float32)], axis=0)

    rows = edge_index[0].astype(jnp.int32)
    cols = edge_index[1].astype(jnp.int32)
    wts = edge_attr.astype(jnp.float32)
    quant = NS * CH
    e_pad = max(((E + quant - 1) // quant) * quant, quant)
    pad = e_pad - E
    rows = jnp.pad(rows, (0, pad), constant_values=N - 1)
    cols = jnp.pad(cols, (0, pad))
    wts = jnp.pad(wts, (0, pad))

    out = _gcn_sc(xw, rows, cols, wts, b, B, N, D_out)
    return out.reshape(B, N, D_out)
